# baseline (speedup 1.0000x reference)
import functools

import jax
import jax.numpy as jnp
from jax import lax
from jax.experimental import pallas as pl
from jax.experimental.pallas import tpu as pltpu
from jax.experimental.pallas import tpu_sc as plsc

EMBED_DIM = 64
BATCH = 16384
HIST = 50
NUM_CARDS = 52


def _make_kernel():
    try:
        info = plsc.get_sparse_core_info()
        nc, ns = info.num_cores, info.num_subcores
    except Exception:
        nc, ns = 2, 16
    nw = nc * ns
    b_per_w = BATCH // nw
    nbb = b_per_w // 128
    ng = b_per_w // 16

    mesh = plsc.VectorSubcoreMesh(
        core_axis_name="c", subcore_axis_name="s", num_cores=nc, num_subcores=ns
    )

    @functools.partial(
        pl.kernel,
        mesh=mesh,
        out_type=jax.ShapeDtypeStruct(
            (HIST, EMBED_DIM // 8, BATCH // 128, 8, 128), jnp.float32
        ),
        scratch_types=[
            pltpu.VMEM((HIST, b_per_w), jnp.int32),
            pltpu.VMEM((2, 8, nbb // 2, 8, 128), jnp.float32),
            pltpu.VMEM((NUM_CARDS * EMBED_DIM,), jnp.float32),
            pltpu.VMEM((EMBED_DIM * NUM_CARDS,), jnp.float32),
            pltpu.VMEM((EMBED_DIM * NUM_CARDS * 16,), jnp.float32),
            pltpu.VMEM((NUM_CARDS, EMBED_DIM), jnp.float32),
            pltpu.VMEM((13, EMBED_DIM), jnp.float32),
            pltpu.VMEM((4, EMBED_DIM), jnp.float32),
            [pltpu.SemaphoreType.DMA] * 2,
        ],
        compiler_params=pltpu.CompilerParams(
            use_tc_tiling_on_sc=False, needs_layout_passes=False
        ),
    )
    def k(
        cards_hbm, card_hbm, rank_hbm, suit_hbm, out_hbm,
        idx_v, tiles_v, fused_v, fused_t, rep_v, card_v, rank_v, suit_v, ssem,
    ):
        wid = lax.axis_index("s") * nc + lax.axis_index("c")
        base = wid * b_per_w

        pltpu.sync_copy(card_hbm, card_v)
        pltpu.sync_copy(rank_hbm, rank_v)
        pltpu.sync_copy(suit_hbm, suit_v)
        pltpu.sync_copy(cards_hbm.at[:, pl.ds(base, b_per_w)], idx_v)
        for c in range(NUM_CARDS):
            for q in range(EMBED_DIM // 16):
                sl = pl.ds(q * 16, 16)
                fused_v[pl.ds(c * EMBED_DIM + q * 16, 16)] = (
                    card_v[c, sl] + rank_v[c // 4, sl] + suit_v[c % 4, sl]
                )
        lanes16 = jnp.arange(16, dtype=jnp.int32)
        for d in range(EMBED_DIM):
            cvec = lanes16 * EMBED_DIM + d
            for c0 in (0, 16, 32, NUM_CARDS - 16):
                vals = plsc.load_gather(fused_v, [cvec + c0 * EMBED_DIM])
                fused_t[pl.ds(d * NUM_CARDS + c0, 16)] = vals

        def rep_body(eb, carry):
            v = fused_t[pl.ds(eb * 16, 16)]
            for j in range(16):
                rep_v[pl.ds((eb * 16 + j) * 16, 16)] = jnp.full(
                    (16,), v[j], dtype=jnp.float32
                )
            return carry

        lax.fori_loop(0, NUM_CARDS * EMBED_DIM // 16, rep_body, 0)

        nbh = nbb // 2

        def issue_scatter(h, half, b):
            for db in range(8):
                pltpu.async_copy(
                    tiles_v.at[b, db],
                    out_hbm.at[h, db, pl.ds(wid * nbb + half * nbh, nbh)],
                    ssem[b],
                )

        def wait_scatter(b):
            for db in range(8):
                pltpu.make_async_copy(
                    tiles_v.at[b, db], out_hbm.at[0, db, pl.ds(0, nbh)], ssem[b]
                ).wait()

        def transpose_into(h, half, b):
            cm16 = [
                idx_v[h, pl.ds(half * (b_per_w // 2) + g * 16, 16)] * 16 + lanes16
                for g in range(ng // 2)
            ]

            @plsc.parallel_loop(0, 8, unroll=4)
            def db_body(db):
                d0 = db * 8
                for ds in range(8):
                    dvec = jnp.full(
                        (16,), (d0 + ds) * NUM_CARDS * 16, dtype=jnp.int32
                    )
                    for bb in range(nbh):
                        for g in range(8):
                            val = plsc.load_gather(
                                rep_v, [cm16[bb * 8 + g] + dvec]
                            )
                            tiles_v[b, db, bb, ds, pl.ds(g * 16, 16)] = val

        def h_body(h, carry):
            for half in range(2):
                b = half
                k2 = h * 2 + half

                @pl.when(k2 >= 2)
                def _drain():
                    wait_scatter(b)

                transpose_into(h, half, b)
                issue_scatter(h, half, b)
            return carry

        lax.fori_loop(0, HIST, h_body, 0)
        wait_scatter(0)
        wait_scatter(1)

    return k


def kernel(cards, card_embed, rank_embed, suit_embed):
    k = _make_kernel()
    cards_t = jnp.transpose(cards.astype(jnp.int32))
    outp = k(cards_t, card_embed, rank_embed, suit_embed)
    return jnp.transpose(outp, (2, 4, 0, 1, 3)).reshape(BATCH, HIST, EMBED_DIM)


# speedup vs baseline: 1.3436x; 1.3436x over previous
import functools

import jax
import jax.numpy as jnp
from jax import lax
from jax.experimental import pallas as pl
from jax.experimental.pallas import tpu as pltpu
from jax.experimental.pallas import tpu_sc as plsc

EMBED_DIM = 64
BATCH = 16384
HIST = 50
NUM_CARDS = 52


def _make_kernel():
    try:
        info = plsc.get_sparse_core_info()
        nc, ns = info.num_cores, info.num_subcores
    except Exception:
        nc, ns = 2, 16
    nw = nc * ns
    b_per_w = BATCH // nw
    nbb = b_per_w // 128
    ng = b_per_w // 16

    mesh = plsc.VectorSubcoreMesh(
        core_axis_name="c", subcore_axis_name="s", num_cores=nc, num_subcores=ns
    )

    @functools.partial(
        pl.kernel,
        mesh=mesh,
        out_type=jax.ShapeDtypeStruct(
            (HIST, EMBED_DIM // 8, BATCH // 128, 8, 128), jnp.float32
        ),
        scratch_types=[
            pltpu.VMEM((HIST, b_per_w), jnp.int32),
            pltpu.VMEM((2, 8, nbb // 2, 8, 128), jnp.float32),
            pltpu.VMEM((NUM_CARDS * EMBED_DIM,), jnp.float32),
            pltpu.VMEM((EMBED_DIM * NUM_CARDS,), jnp.float32),
            pltpu.VMEM((EMBED_DIM * NUM_CARDS * 16,), jnp.float32),
            pltpu.VMEM((NUM_CARDS, EMBED_DIM), jnp.float32),
            pltpu.VMEM((13, EMBED_DIM), jnp.float32),
            pltpu.VMEM((4, EMBED_DIM), jnp.float32),
            [pltpu.SemaphoreType.DMA] * 2,
        ],
        compiler_params=pltpu.CompilerParams(
            use_tc_tiling_on_sc=False, needs_layout_passes=False
        ),
    )
    def k(
        cards_hbm, card_hbm, rank_hbm, suit_hbm, out_hbm,
        idx_v, tiles_v, fused_v, fused_t, rep_v, card_v, rank_v, suit_v, ssem,
    ):
        wid = lax.axis_index("s") * nc + lax.axis_index("c")
        base = wid * b_per_w

        pltpu.sync_copy(card_hbm, card_v)
        pltpu.sync_copy(rank_hbm, rank_v)
        pltpu.sync_copy(suit_hbm, suit_v)
        pltpu.sync_copy(cards_hbm.at[:, pl.ds(base, b_per_w)], idx_v)
        for c in range(NUM_CARDS):
            for q in range(EMBED_DIM // 16):
                sl = pl.ds(q * 16, 16)
                fused_v[pl.ds(c * EMBED_DIM + q * 16, 16)] = (
                    card_v[c, sl] + rank_v[c // 4, sl] + suit_v[c % 4, sl]
                )
        lanes16 = jnp.arange(16, dtype=jnp.int32)
        for d in range(EMBED_DIM):
            cvec = lanes16 * EMBED_DIM + d
            for c0 in (0, 16, 32, NUM_CARDS - 16):
                vals = plsc.load_gather(fused_v, [cvec + c0 * EMBED_DIM])
                fused_t[pl.ds(d * NUM_CARDS + c0, 16)] = vals

        def rep_body(eb, carry):
            v = fused_t[pl.ds(eb * 16, 16)]
            for j in range(16):
                rep_v[pl.ds((eb * 16 + j) * 16, 16)] = jnp.full(
                    (16,), v[j], dtype=jnp.float32
                )
            return carry

        lax.fori_loop(0, NUM_CARDS * EMBED_DIM // 16, rep_body, 0)

        nbh = nbb // 2

        def issue_scatter(h, half, b):
            for db in range(8):
                pltpu.async_copy(
                    tiles_v.at[b, db],
                    out_hbm.at[h, db, pl.ds(wid * nbb + half * nbh, nbh)],
                    ssem[b],
                )

        def wait_scatter(b):
            for db in range(8):
                pltpu.make_async_copy(
                    tiles_v.at[b, db], out_hbm.at[0, db, pl.ds(0, nbh)], ssem[b]
                ).wait()

        def transpose_into(h, half, b):
            cm16 = [
                idx_v[h, pl.ds(half * (b_per_w // 2) + g * 16, 16)] * 16 + lanes16
                for g in range(ng // 2)
            ]

            @plsc.parallel_loop(0, 8, unroll=2)
            def db_body(db):
                d0 = db * 8
                for ds in range(8):
                    dvec = jnp.full(
                        (16,), (d0 + ds) * NUM_CARDS * 16, dtype=jnp.int32
                    )
                    for bb in range(nbh):
                        for g in range(8):
                            val = plsc.load_gather(
                                rep_v, [cm16[bb * 8 + g] + dvec]
                            )
                            tiles_v[b, db, bb, ds, pl.ds(g * 16, 16)] = val

        def h_body(h, carry):
            for half in range(2):
                b = half
                k2 = h * 2 + half

                @pl.when(k2 >= 2)
                def _drain():
                    wait_scatter(b)

                transpose_into(h, half, b)
                issue_scatter(h, half, b)
            return carry

        lax.fori_loop(0, HIST, h_body, 0)
        wait_scatter(0)
        wait_scatter(1)

    return k


def kernel(cards, card_embed, rank_embed, suit_embed):
    k = _make_kernel()
    cards_t = jnp.transpose(cards.astype(jnp.int32))
    outp = k(cards_t, card_embed, rank_embed, suit_embed)
    return jnp.transpose(outp, (2, 4, 0, 1, 3)).reshape(BATCH, HIST, EMBED_DIM)
